# Initial kernel scaffold; baseline (speedup 1.0000x reference)
import jax
import jax.numpy as jnp
from jax.experimental import pallas as pl


def kernel(x, edge_index, W1, b1, W2, b2, Wmu, bmu, Wlv, blv):
    raise NotImplementedError("write your pallas kernel here")


# speedup vs baseline: 19.0816x; 19.0816x over previous
import functools

import jax
import jax.numpy as jnp
from jax import lax
from jax.experimental import pallas as pl
from jax.experimental.pallas import tpu as pltpu
from jax.experimental.pallas import tpu_sc as plsc

N = 10000
D = 128
E = 320000
LAT = 64
NC = 2
NS = 16
NW = NC * NS
EPW = E // NW
K = 80
CH = EPW // K
NP = 10240
RPT = NP // NS
DP = 10240
DW = 16
DS = DP // NS

_mesh = plsc.VectorSubcoreMesh(core_axis_name="c", subcore_axis_name="s")


def _deg_body(dst_hbm, out_hbm, dst_v, buf_v, acc_sh):
    c = lax.axis_index("c")
    s = lax.axis_index("s")
    wid = s * NC + c
    zero16 = jnp.zeros((DW,), jnp.float32)
    for r in range(K):
        buf_v[r] = zero16
    for i in range(DS // K):
        pltpu.sync_copy(buf_v, acc_sh.at[pl.ds(s * DS + i * K, K)])
    one16 = jnp.ones((DW,), jnp.float32)
    for r in range(K):
        buf_v[r] = one16
    pltpu.sync_copy(dst_hbm.at[wid], dst_v)
    plsc.subcore_barrier()

    @pl.loop(0, CH)
    def _(j):
        pltpu.sync_copy(buf_v, acc_sh.at[dst_v.at[j]], add=True)

    plsc.subcore_barrier()
    pltpu.sync_copy(acc_sh.at[pl.ds(s * DS, DS)], out_hbm.at[c, pl.ds(s * DS, DS)])


_deg_call = functools.partial(
    pl.kernel,
    out_type=jax.ShapeDtypeStruct((NC, DP, DW), jnp.float32),
    mesh=_mesh,
    scratch_types=[
        pltpu.VMEM((CH, K), jnp.int32),
        pltpu.VMEM((K, DW), jnp.float32),
        pltpu.VMEM_SHARED((DP, DW), jnp.float32),
    ],
)(_deg_body)


def _rows_body(hp_hbm, src_hbm, dst_hbm, out_hbm, src_v, dst_v, rows_v, acc_sh, sem):
    c = lax.axis_index("c")
    s = lax.axis_index("s")
    wid = s * NC + c
    @pl.when(s < NS - 1)
    def _init_full():
        pltpu.sync_copy(hp_hbm.at[pl.ds(s * RPT, RPT)], acc_sh.at[pl.ds(s * RPT, RPT)])

    @pl.when(s == NS - 1)
    def _init_tail():
        pltpu.sync_copy(hp_hbm.at[pl.ds((NS - 1) * RPT, N - (NS - 1) * RPT)],
                        acc_sh.at[pl.ds((NS - 1) * RPT, N - (NS - 1) * RPT)])
    pltpu.sync_copy(src_hbm.at[wid], src_v)
    pltpu.sync_copy(dst_hbm.at[wid], dst_v)
    plsc.subcore_barrier()

    @pl.loop(0, CH)
    def _(j):
        pltpu.async_copy(hp_hbm.at[src_v.at[j]], rows_v, sem).wait()
        pltpu.sync_copy(rows_v, acc_sh.at[dst_v.at[j]], add=True)

    plsc.subcore_barrier()
    pltpu.sync_copy(acc_sh.at[pl.ds(s * RPT, RPT)], out_hbm.at[c, pl.ds(s * RPT, RPT)])


_rows_call = functools.partial(
    pl.kernel,
    out_type=jax.ShapeDtypeStruct((NC, NP, D), jnp.float32),
    mesh=_mesh,
    scratch_types=[
        pltpu.VMEM((CH, K), jnp.int32),
        pltpu.VMEM((CH, K), jnp.int32),
        pltpu.VMEM((K, D), jnp.float32),
        pltpu.VMEM_SHARED((NP, D), jnp.float32),
        pltpu.SemaphoreType.DMA,
    ],
)(_rows_body)


BLK = 2000
GRID = N // BLK


def _tc1_body(x_ref, w1_ref, degp_ref, hp_ref, dinv_ref):
    deg = degp_ref[0, :, 0:1] + degp_ref[1, :, 0:1] + 1.0
    dinv = lax.rsqrt(deg)
    z = jnp.dot(x_ref[...], w1_ref[...], preferred_element_type=jnp.float32)
    hp_ref[...] = z * dinv
    dinv_ref[...] = dinv


def _tc1(x, W1, degp):
    return pl.pallas_call(
        _tc1_body,
        grid=(GRID,),
        in_specs=[
            pl.BlockSpec((BLK, D), lambda i: (i, 0)),
            pl.BlockSpec((D, D), lambda i: (0, 0)),
            pl.BlockSpec((NC, BLK, DW), lambda i: (0, i, 0)),
        ],
        out_specs=[
            pl.BlockSpec((BLK, D), lambda i: (i, 0)),
            pl.BlockSpec((BLK, 1), lambda i: (i, 0)),
        ],
        out_shape=[
            jax.ShapeDtypeStruct((N, D), jnp.float32),
            jax.ShapeDtypeStruct((N, 1), jnp.float32),
        ],
    )(x, W1, degp)


def _tc2_body(s_ref, hp1_ref, dinv_ref, b1_ref, w2_ref, hp2_ref):
    dinv = dinv_ref[...]
    agg = (s_ref[0] + s_ref[1] - hp1_ref[...]) * dinv + b1_ref[...]
    h1 = jnp.maximum(agg, 0.0)
    z2 = jnp.dot(h1, w2_ref[...], preferred_element_type=jnp.float32)
    hp2_ref[...] = z2 * dinv


def _tc2(S1, hp1, dinv, b1, W2):
    return pl.pallas_call(
        _tc2_body,
        grid=(GRID,),
        in_specs=[
            pl.BlockSpec((NC, BLK, D), lambda i: (0, i, 0)),
            pl.BlockSpec((BLK, D), lambda i: (i, 0)),
            pl.BlockSpec((BLK, 1), lambda i: (i, 0)),
            pl.BlockSpec((1, D), lambda i: (0, 0)),
            pl.BlockSpec((D, D), lambda i: (0, 0)),
        ],
        out_specs=pl.BlockSpec((BLK, D), lambda i: (i, 0)),
        out_shape=jax.ShapeDtypeStruct((N, D), jnp.float32),
    )(S1, hp1, dinv, b1, W2)


def _tc3_body(s_ref, hp2_ref, dinv_ref, b2_ref, wmu_ref, bmu_ref, wlv_ref,
              blv_ref, mu_ref, lv_ref, gacc):
    i = pl.program_id(0)

    @pl.when(i == 0)
    def _init():
        gacc[...] = jnp.zeros_like(gacc)

    dinv = dinv_ref[...]
    agg = (s_ref[0] + s_ref[1] - hp2_ref[...]) * dinv + b2_ref[...]
    h2 = jnp.maximum(agg, 0.0)
    gacc[...] += jnp.sum(h2, axis=0, keepdims=True)

    @pl.when(i == pl.num_programs(0) - 1)
    def _fin():
        g = gacc[...] * (1.0 / N)
        mu_ref[...] = jnp.dot(g, wmu_ref[...], preferred_element_type=jnp.float32) + bmu_ref[...]
        lv_ref[...] = jnp.dot(g, wlv_ref[...], preferred_element_type=jnp.float32) + blv_ref[...]


def _tc3(S2, hp2, dinv, b2, Wmu, bmu, Wlv, blv):
    return pl.pallas_call(
        _tc3_body,
        grid=(GRID,),
        in_specs=[
            pl.BlockSpec((NC, BLK, D), lambda i: (0, i, 0)),
            pl.BlockSpec((BLK, D), lambda i: (i, 0)),
            pl.BlockSpec((BLK, 1), lambda i: (i, 0)),
            pl.BlockSpec((1, D), lambda i: (0, 0)),
            pl.BlockSpec((D, LAT), lambda i: (0, 0)),
            pl.BlockSpec((1, LAT), lambda i: (0, 0)),
            pl.BlockSpec((D, LAT), lambda i: (0, 0)),
            pl.BlockSpec((1, LAT), lambda i: (0, 0)),
        ],
        out_specs=[
            pl.BlockSpec((1, LAT), lambda i: (0, 0)),
            pl.BlockSpec((1, LAT), lambda i: (0, 0)),
        ],
        out_shape=[
            jax.ShapeDtypeStruct((1, LAT), jnp.float32),
            jax.ShapeDtypeStruct((1, LAT), jnp.float32),
        ],
        scratch_shapes=[pltpu.VMEM((1, D), jnp.float32)],
    )(S2, hp2, dinv, b2, Wmu, bmu, Wlv, blv)


def kernel(x, edge_index, W1, b1, W2, b2, Wmu, bmu, Wlv, blv):
    ei = edge_index.astype(jnp.int32)
    src3 = ei[0].reshape(NW, CH, K)
    dst3 = ei[1].reshape(NW, CH, K)

    degp = _deg_call(dst3)[:, :N]
    hp1, dinv = _tc1(x, W1, degp)
    S1 = _rows_call(hp1, src3, dst3)[:, :N]
    hp2 = _tc2(S1, hp1, dinv, b1.reshape(1, D), W2)
    S2 = _rows_call(hp2, src3, dst3)[:, :N]
    mu, lv = _tc3(S2, hp2, dinv, b2.reshape(1, D), Wmu,
                  bmu.reshape(1, LAT), Wlv, blv.reshape(1, LAT))
    return mu.reshape(LAT), lv.reshape(LAT)


# speedup vs baseline: 27.1644x; 1.4236x over previous
import functools

import jax
import jax.numpy as jnp
from jax import lax
from jax.experimental import pallas as pl
from jax.experimental.pallas import tpu as pltpu
from jax.experimental.pallas import tpu_sc as plsc

N = 10000
D = 128
E = 320000
LAT = 64
NC = 2
NS = 16
NW = NC * NS
EPW = E // NW
K = 80
CH = EPW // K
GR = 5
CHG = CH // GR
NP = 10240
RPT = NP // NS
DP = 10240
DW = 16
DS = DP // NS

_mesh = plsc.VectorSubcoreMesh(core_axis_name="c", subcore_axis_name="s")


def _deg_body(dst_hbm, out_hbm, dst_v, buf_v, acc_sh):
    c = lax.axis_index("c")
    s = lax.axis_index("s")
    wid = s * NC + c
    zero16 = jnp.zeros((DW,), jnp.float32)
    for r in range(K):
        buf_v[r] = zero16
    for i in range(DS // K):
        pltpu.sync_copy(buf_v, acc_sh.at[pl.ds(s * DS + i * K, K)])
    one16 = jnp.ones((DW,), jnp.float32)
    for r in range(K):
        buf_v[r] = one16
    pltpu.sync_copy(dst_hbm.at[wid], dst_v)
    plsc.subcore_barrier()

    @pl.loop(0, CH)
    def _(j):
        pltpu.sync_copy(buf_v, acc_sh.at[dst_v.at[j]], add=True)

    plsc.subcore_barrier()
    pltpu.sync_copy(acc_sh.at[pl.ds(s * DS, DS)], out_hbm.at[c, pl.ds(s * DS, DS)])


_deg_call = functools.partial(
    pl.kernel,
    out_type=jax.ShapeDtypeStruct((NC, DP, DW), jnp.float32),
    mesh=_mesh,
    scratch_types=[
        pltpu.VMEM((CH, K), jnp.int32),
        pltpu.VMEM((K, DW), jnp.float32),
        pltpu.VMEM_SHARED((DP, DW), jnp.float32),
    ],
)(_deg_body)


def _rows_body(hp_hbm, src_hbm, dst_hbm, out_hbm, src_v, dst_v, rows0_v, rows1_v,
               acc_sh, sem0, sem1):
    c = lax.axis_index("c")
    s = lax.axis_index("s")
    wid = s * NC + c
    @pl.when(s < NS - 1)
    def _init_full():
        pltpu.sync_copy(hp_hbm.at[pl.ds(s * RPT, RPT)], acc_sh.at[pl.ds(s * RPT, RPT)])

    @pl.when(s == NS - 1)
    def _init_tail():
        pltpu.sync_copy(hp_hbm.at[pl.ds((NS - 1) * RPT, N - (NS - 1) * RPT)],
                        acc_sh.at[pl.ds((NS - 1) * RPT, N - (NS - 1) * RPT)])
    plsc.subcore_barrier()

    for g in range(GR):
        pltpu.sync_copy(src_hbm.at[wid, g], src_v)
        pltpu.sync_copy(dst_hbm.at[wid, g], dst_v)
        pltpu.async_copy(hp_hbm.at[src_v.at[0]], rows0_v, sem0)

        @pl.loop(0, CHG, step=2)
        def _(j):
            @pl.when(j + 1 < CHG)
            def _g1():
                pltpu.async_copy(hp_hbm.at[src_v.at[j + 1]], rows1_v, sem1)

            pltpu.make_async_copy(hp_hbm.at[src_v.at[j]], rows0_v, sem0).wait()
            pltpu.sync_copy(rows0_v, acc_sh.at[dst_v.at[j]], add=True)

            @pl.when(j + 2 < CHG)
            def _g0():
                pltpu.async_copy(hp_hbm.at[src_v.at[j + 2]], rows0_v, sem0)

            @pl.when(j + 1 < CHG)
            def _s1():
                pltpu.make_async_copy(hp_hbm.at[src_v.at[j + 1]], rows1_v, sem1).wait()
                pltpu.sync_copy(rows1_v, acc_sh.at[dst_v.at[j + 1]], add=True)

    plsc.subcore_barrier()
    pltpu.sync_copy(acc_sh.at[pl.ds(s * RPT, RPT)], out_hbm.at[c, pl.ds(s * RPT, RPT)])


_rows_call = functools.partial(
    pl.kernel,
    out_type=jax.ShapeDtypeStruct((NC, NP, D), jnp.float32),
    mesh=_mesh,
    scratch_types=[
        pltpu.VMEM((CHG, K), jnp.int32),
        pltpu.VMEM((CHG, K), jnp.int32),
        pltpu.VMEM((K, D), jnp.float32),
        pltpu.VMEM((K, D), jnp.float32),
        pltpu.VMEM_SHARED((NP, D), jnp.float32),
        pltpu.SemaphoreType.DMA,
        pltpu.SemaphoreType.DMA,
    ],
)(_rows_body)


BLK = 2000
GRID = N // BLK


def _tc1_body(x_ref, w1_ref, degp_ref, hp_ref, dinv_ref):
    deg = degp_ref[0, :, 0:1] + degp_ref[1, :, 0:1] + 1.0
    dinv = lax.rsqrt(deg)
    z = jnp.dot(x_ref[...], w1_ref[...], preferred_element_type=jnp.float32)
    hp_ref[...] = z * dinv
    dinv_ref[...] = dinv


def _tc1(x, W1, degp):
    return pl.pallas_call(
        _tc1_body,
        grid=(GRID,),
        in_specs=[
            pl.BlockSpec((BLK, D), lambda i: (i, 0)),
            pl.BlockSpec((D, D), lambda i: (0, 0)),
            pl.BlockSpec((NC, BLK, DW), lambda i: (0, i, 0)),
        ],
        out_specs=[
            pl.BlockSpec((BLK, D), lambda i: (i, 0)),
            pl.BlockSpec((BLK, 1), lambda i: (i, 0)),
        ],
        out_shape=[
            jax.ShapeDtypeStruct((N, D), jnp.float32),
            jax.ShapeDtypeStruct((N, 1), jnp.float32),
        ],
    )(x, W1, degp)


def _tc2_body(s_ref, hp1_ref, dinv_ref, b1_ref, w2_ref, hp2_ref):
    dinv = dinv_ref[...]
    agg = (s_ref[0] + s_ref[1] - hp1_ref[...]) * dinv + b1_ref[...]
    h1 = jnp.maximum(agg, 0.0)
    z2 = jnp.dot(h1, w2_ref[...], preferred_element_type=jnp.float32)
    hp2_ref[...] = z2 * dinv


def _tc2(S1, hp1, dinv, b1, W2):
    return pl.pallas_call(
        _tc2_body,
        grid=(GRID,),
        in_specs=[
            pl.BlockSpec((NC, BLK, D), lambda i: (0, i, 0)),
            pl.BlockSpec((BLK, D), lambda i: (i, 0)),
            pl.BlockSpec((BLK, 1), lambda i: (i, 0)),
            pl.BlockSpec((1, D), lambda i: (0, 0)),
            pl.BlockSpec((D, D), lambda i: (0, 0)),
        ],
        out_specs=pl.BlockSpec((BLK, D), lambda i: (i, 0)),
        out_shape=jax.ShapeDtypeStruct((N, D), jnp.float32),
    )(S1, hp1, dinv, b1, W2)


def _tc3_body(s_ref, hp2_ref, dinv_ref, b2_ref, wmu_ref, bmu_ref, wlv_ref,
              blv_ref, mu_ref, lv_ref, gacc):
    i = pl.program_id(0)

    @pl.when(i == 0)
    def _init():
        gacc[...] = jnp.zeros_like(gacc)

    dinv = dinv_ref[...]
    agg = (s_ref[0] + s_ref[1] - hp2_ref[...]) * dinv + b2_ref[...]
    h2 = jnp.maximum(agg, 0.0)
    gacc[...] += jnp.sum(h2, axis=0, keepdims=True)

    @pl.when(i == pl.num_programs(0) - 1)
    def _fin():
        g = gacc[...] * (1.0 / N)
        mu_ref[...] = jnp.dot(g, wmu_ref[...], preferred_element_type=jnp.float32) + bmu_ref[...]
        lv_ref[...] = jnp.dot(g, wlv_ref[...], preferred_element_type=jnp.float32) + blv_ref[...]


def _tc3(S2, hp2, dinv, b2, Wmu, bmu, Wlv, blv):
    return pl.pallas_call(
        _tc3_body,
        grid=(GRID,),
        in_specs=[
            pl.BlockSpec((NC, BLK, D), lambda i: (0, i, 0)),
            pl.BlockSpec((BLK, D), lambda i: (i, 0)),
            pl.BlockSpec((BLK, 1), lambda i: (i, 0)),
            pl.BlockSpec((1, D), lambda i: (0, 0)),
            pl.BlockSpec((D, LAT), lambda i: (0, 0)),
            pl.BlockSpec((1, LAT), lambda i: (0, 0)),
            pl.BlockSpec((D, LAT), lambda i: (0, 0)),
            pl.BlockSpec((1, LAT), lambda i: (0, 0)),
        ],
        out_specs=[
            pl.BlockSpec((1, LAT), lambda i: (0, 0)),
            pl.BlockSpec((1, LAT), lambda i: (0, 0)),
        ],
        out_shape=[
            jax.ShapeDtypeStruct((1, LAT), jnp.float32),
            jax.ShapeDtypeStruct((1, LAT), jnp.float32),
        ],
        scratch_shapes=[pltpu.VMEM((1, D), jnp.float32)],
    )(S2, hp2, dinv, b2, Wmu, bmu, Wlv, blv)


def kernel(x, edge_index, W1, b1, W2, b2, Wmu, bmu, Wlv, blv):
    ei = edge_index.astype(jnp.int32)
    src3 = ei[0].reshape(NW, CH, K)
    dst3 = ei[1].reshape(NW, CH, K)
    src4 = ei[0].reshape(NW, GR, CHG, K)
    dst4 = ei[1].reshape(NW, GR, CHG, K)

    degp = _deg_call(dst3)[:, :N]
    hp1, dinv = _tc1(x, W1, degp)
    S1 = _rows_call(hp1, src4, dst4)[:, :N]
    hp2 = _tc2(S1, hp1, dinv, b1.reshape(1, D), W2)
    S2 = _rows_call(hp2, src4, dst4)[:, :N]
    mu, lv = _tc3(S2, hp2, dinv, b2.reshape(1, D), Wmu,
                  bmu.reshape(1, LAT), Wlv, blv.reshape(1, LAT))
    return mu.reshape(LAT), lv.reshape(LAT)


# speedup vs baseline: 27.9452x; 1.0287x over previous
import functools

import jax
import jax.numpy as jnp
from jax import lax
from jax.experimental import pallas as pl
from jax.experimental.pallas import tpu as pltpu
from jax.experimental.pallas import tpu_sc as plsc

N = 10000
D = 128
E = 320000
LAT = 64
NC = 2
NS = 16
NW = NC * NS
EPW = E // NW
K = 80
CH = EPW // K
GR = 5
CHG = CH // GR
NP = 10240
RPT = NP // NS
DP = 10240
DW = 16
DS = DP // NS

_mesh = plsc.VectorSubcoreMesh(core_axis_name="c", subcore_axis_name="s")


def _deg_body(dst_hbm, out_hbm, dst_v, buf_v, acc_sh):
    c = lax.axis_index("c")
    s = lax.axis_index("s")
    wid = s * NC + c
    zero16 = jnp.zeros((DW,), jnp.float32)
    for r in range(K):
        buf_v[r] = zero16
    for i in range(DS // K):
        pltpu.sync_copy(buf_v, acc_sh.at[pl.ds(s * DS + i * K, K)])
    one16 = jnp.ones((DW,), jnp.float32)
    for r in range(K):
        buf_v[r] = one16
    pltpu.sync_copy(dst_hbm.at[wid], dst_v)
    plsc.subcore_barrier()

    @pl.loop(0, CH)
    def _(j):
        pltpu.sync_copy(buf_v, acc_sh.at[dst_v.at[j]], add=True)

    plsc.subcore_barrier()
    pltpu.sync_copy(acc_sh.at[pl.ds(s * DS, DS)], out_hbm.at[c, pl.ds(s * DS, DS)])


_deg_call = functools.partial(
    pl.kernel,
    out_type=jax.ShapeDtypeStruct((NC, DP, DW), jnp.float32),
    mesh=_mesh,
    scratch_types=[
        pltpu.VMEM((CH, K), jnp.int32),
        pltpu.VMEM((K, DW), jnp.float32),
        pltpu.VMEM_SHARED((DP, DW), jnp.float32),
    ],
)(_deg_body)


def _rows_body(hp_hbm, src_hbm, dst_hbm, out_hbm, srcA_v, dstA_v, srcB_v, dstB_v,
               rows0_v, rows1_v, acc_sh, sem0, sem1, semi):
    c = lax.axis_index("c")
    s = lax.axis_index("s")
    wid = s * NC + c
    @pl.when(s < NS - 1)
    def _init_full():
        pltpu.sync_copy(hp_hbm.at[pl.ds(s * RPT, RPT)], acc_sh.at[pl.ds(s * RPT, RPT)])

    @pl.when(s == NS - 1)
    def _init_tail():
        pltpu.sync_copy(hp_hbm.at[pl.ds((NS - 1) * RPT, N - (NS - 1) * RPT)],
                        acc_sh.at[pl.ds((NS - 1) * RPT, N - (NS - 1) * RPT)])
    plsc.subcore_barrier()

    idx_bufs = [(srcA_v, dstA_v), (srcB_v, dstB_v)]
    pltpu.async_copy(src_hbm.at[wid, 0], srcA_v, semi)
    pltpu.async_copy(dst_hbm.at[wid, 0], dstA_v, semi)
    for g in range(GR):
        src_v, dst_v = idx_bufs[g % 2]
        pltpu.make_async_copy(src_hbm.at[wid, g], src_v, semi).wait()
        pltpu.make_async_copy(dst_hbm.at[wid, g], dst_v, semi).wait()
        if g + 1 < GR:
            nsrc_v, ndst_v = idx_bufs[(g + 1) % 2]
            pltpu.async_copy(src_hbm.at[wid, g + 1], nsrc_v, semi)
            pltpu.async_copy(dst_hbm.at[wid, g + 1], ndst_v, semi)
        pltpu.async_copy(hp_hbm.at[src_v.at[0]], rows0_v, sem0)

        @pl.loop(0, CHG, step=2)
        def _(j):
            @pl.when(j + 1 < CHG)
            def _g1():
                pltpu.async_copy(hp_hbm.at[src_v.at[j + 1]], rows1_v, sem1)

            pltpu.make_async_copy(hp_hbm.at[src_v.at[j]], rows0_v, sem0).wait()
            pltpu.sync_copy(rows0_v, acc_sh.at[dst_v.at[j]], add=True)

            @pl.when(j + 2 < CHG)
            def _g0():
                pltpu.async_copy(hp_hbm.at[src_v.at[j + 2]], rows0_v, sem0)

            @pl.when(j + 1 < CHG)
            def _s1():
                pltpu.make_async_copy(hp_hbm.at[src_v.at[j + 1]], rows1_v, sem1).wait()
                pltpu.sync_copy(rows1_v, acc_sh.at[dst_v.at[j + 1]], add=True)

    plsc.subcore_barrier()
    pltpu.sync_copy(acc_sh.at[pl.ds(s * RPT, RPT)], out_hbm.at[c, pl.ds(s * RPT, RPT)])


_rows_call = functools.partial(
    pl.kernel,
    out_type=jax.ShapeDtypeStruct((NC, NP, D), jnp.float32),
    mesh=_mesh,
    scratch_types=[
        pltpu.VMEM((CHG, K), jnp.int32),
        pltpu.VMEM((CHG, K), jnp.int32),
        pltpu.VMEM((CHG, K), jnp.int32),
        pltpu.VMEM((CHG, K), jnp.int32),
        pltpu.VMEM((K, D), jnp.float32),
        pltpu.VMEM((K, D), jnp.float32),
        pltpu.VMEM_SHARED((NP, D), jnp.float32),
        pltpu.SemaphoreType.DMA,
        pltpu.SemaphoreType.DMA,
        pltpu.SemaphoreType.DMA,
    ],
)(_rows_body)


BLK = 2000
GRID = N // BLK


def _tc1_body(x_ref, w1_ref, degp_ref, hp_ref, dinv_ref):
    deg = degp_ref[0, :, 0:1] + degp_ref[1, :, 0:1] + 1.0
    dinv = lax.rsqrt(deg)
    z = jnp.dot(x_ref[...], w1_ref[...], preferred_element_type=jnp.float32)
    hp_ref[...] = z * dinv
    dinv_ref[...] = dinv


def _tc1(x, W1, degp):
    return pl.pallas_call(
        _tc1_body,
        grid=(GRID,),
        in_specs=[
            pl.BlockSpec((BLK, D), lambda i: (i, 0)),
            pl.BlockSpec((D, D), lambda i: (0, 0)),
            pl.BlockSpec((NC, BLK, DW), lambda i: (0, i, 0)),
        ],
        out_specs=[
            pl.BlockSpec((BLK, D), lambda i: (i, 0)),
            pl.BlockSpec((BLK, 1), lambda i: (i, 0)),
        ],
        out_shape=[
            jax.ShapeDtypeStruct((N, D), jnp.float32),
            jax.ShapeDtypeStruct((N, 1), jnp.float32),
        ],
    )(x, W1, degp)


def _tc2_body(s_ref, hp1_ref, dinv_ref, b1_ref, w2_ref, hp2_ref):
    dinv = dinv_ref[...]
    agg = (s_ref[0] + s_ref[1] - hp1_ref[...]) * dinv + b1_ref[...]
    h1 = jnp.maximum(agg, 0.0)
    z2 = jnp.dot(h1, w2_ref[...], preferred_element_type=jnp.float32)
    hp2_ref[...] = z2 * dinv


def _tc2(S1, hp1, dinv, b1, W2):
    return pl.pallas_call(
        _tc2_body,
        grid=(GRID,),
        in_specs=[
            pl.BlockSpec((NC, BLK, D), lambda i: (0, i, 0)),
            pl.BlockSpec((BLK, D), lambda i: (i, 0)),
            pl.BlockSpec((BLK, 1), lambda i: (i, 0)),
            pl.BlockSpec((1, D), lambda i: (0, 0)),
            pl.BlockSpec((D, D), lambda i: (0, 0)),
        ],
        out_specs=pl.BlockSpec((BLK, D), lambda i: (i, 0)),
        out_shape=jax.ShapeDtypeStruct((N, D), jnp.float32),
    )(S1, hp1, dinv, b1, W2)


def _tc3_body(s_ref, hp2_ref, dinv_ref, b2_ref, wmu_ref, bmu_ref, wlv_ref,
              blv_ref, mu_ref, lv_ref, gacc):
    i = pl.program_id(0)

    @pl.when(i == 0)
    def _init():
        gacc[...] = jnp.zeros_like(gacc)

    dinv = dinv_ref[...]
    agg = (s_ref[0] + s_ref[1] - hp2_ref[...]) * dinv + b2_ref[...]
    h2 = jnp.maximum(agg, 0.0)
    gacc[...] += jnp.sum(h2, axis=0, keepdims=True)

    @pl.when(i == pl.num_programs(0) - 1)
    def _fin():
        g = gacc[...] * (1.0 / N)
        mu_ref[...] = jnp.dot(g, wmu_ref[...], preferred_element_type=jnp.float32) + bmu_ref[...]
        lv_ref[...] = jnp.dot(g, wlv_ref[...], preferred_element_type=jnp.float32) + blv_ref[...]


def _tc3(S2, hp2, dinv, b2, Wmu, bmu, Wlv, blv):
    return pl.pallas_call(
        _tc3_body,
        grid=(GRID,),
        in_specs=[
            pl.BlockSpec((NC, BLK, D), lambda i: (0, i, 0)),
            pl.BlockSpec((BLK, D), lambda i: (i, 0)),
            pl.BlockSpec((BLK, 1), lambda i: (i, 0)),
            pl.BlockSpec((1, D), lambda i: (0, 0)),
            pl.BlockSpec((D, LAT), lambda i: (0, 0)),
            pl.BlockSpec((1, LAT), lambda i: (0, 0)),
            pl.BlockSpec((D, LAT), lambda i: (0, 0)),
            pl.BlockSpec((1, LAT), lambda i: (0, 0)),
        ],
        out_specs=[
            pl.BlockSpec((1, LAT), lambda i: (0, 0)),
            pl.BlockSpec((1, LAT), lambda i: (0, 0)),
        ],
        out_shape=[
            jax.ShapeDtypeStruct((1, LAT), jnp.float32),
            jax.ShapeDtypeStruct((1, LAT), jnp.float32),
        ],
        scratch_shapes=[pltpu.VMEM((1, D), jnp.float32)],
    )(S2, hp2, dinv, b2, Wmu, bmu, Wlv, blv)


def kernel(x, edge_index, W1, b1, W2, b2, Wmu, bmu, Wlv, blv):
    ei = edge_index.astype(jnp.int32)
    src3 = ei[0].reshape(NW, CH, K)
    dst3 = ei[1].reshape(NW, CH, K)
    src4 = ei[0].reshape(NW, GR, CHG, K)
    dst4 = ei[1].reshape(NW, GR, CHG, K)

    degp = _deg_call(dst3)[:, :N]
    hp1, dinv = _tc1(x, W1, degp)
    S1 = _rows_call(hp1, src4, dst4)[:, :N]
    hp2 = _tc2(S1, hp1, dinv, b1.reshape(1, D), W2)
    S2 = _rows_call(hp2, src4, dst4)[:, :N]
    mu, lv = _tc3(S2, hp2, dinv, b2.reshape(1, D), Wmu,
                  bmu.reshape(1, LAT), Wlv, blv.reshape(1, LAT))
    return mu.reshape(LAT), lv.reshape(LAT)


# speedup vs baseline: 29.7477x; 1.0645x over previous
import functools

import jax
import jax.numpy as jnp
from jax import lax
from jax.experimental import pallas as pl
from jax.experimental.pallas import tpu as pltpu
from jax.experimental.pallas import tpu_sc as plsc

N = 10000
D = 128
E = 320000
LAT = 64
NC = 2
NS = 16
NW = NC * NS
EPW = E // NW
K = 80
CH = EPW // K
GR = 5
CHG = CH // GR
NP = 10240
RPT = NP // NS
DP = 10240
DW = 16
DS = DP // NS

_mesh = plsc.VectorSubcoreMesh(core_axis_name="c", subcore_axis_name="s")


def _deg_body(dst_hbm, out_hbm, dst_v, buf_v, acc_sh):
    c = lax.axis_index("c")
    s = lax.axis_index("s")
    wid = s * NC + c
    zero16 = jnp.zeros((DW,), jnp.float32)
    for r in range(K):
        buf_v[r] = zero16
    for i in range(DS // K):
        pltpu.sync_copy(buf_v, acc_sh.at[pl.ds(s * DS + i * K, K)])
    one16 = jnp.ones((DW,), jnp.float32)
    for r in range(K):
        buf_v[r] = one16
    pltpu.sync_copy(dst_hbm.at[wid], dst_v)
    plsc.subcore_barrier()

    @pl.loop(0, CH)
    def _(j):
        pltpu.sync_copy(buf_v, acc_sh.at[dst_v.at[j]], add=True)

    plsc.subcore_barrier()
    pltpu.sync_copy(acc_sh.at[pl.ds(s * DS, DS)], out_hbm.at[c, pl.ds(s * DS, DS)])


_deg_call = functools.partial(
    pl.kernel,
    out_type=jax.ShapeDtypeStruct((NC, DP, DW), jnp.float32),
    mesh=_mesh,
    scratch_types=[
        pltpu.VMEM((CH, K), jnp.int32),
        pltpu.VMEM((K, DW), jnp.float32),
        pltpu.VMEM_SHARED((DP, DW), jnp.float32),
    ],
)(_deg_body)


def _rows_body(hp_hbm, src_hbm, dst_hbm, out_hbm, srcA_v, dstA_v, srcB_v, dstB_v,
               rows0_v, rows1_v, acc_sh, sem0, sem1, semi):
    c = lax.axis_index("c")
    s = lax.axis_index("s")
    wid = s * NC + c
    @pl.when(s < NS - 1)
    def _init_full():
        pltpu.sync_copy(hp_hbm.at[pl.ds(s * RPT, RPT)], acc_sh.at[pl.ds(s * RPT, RPT)])

    @pl.when(s == NS - 1)
    def _init_tail():
        pltpu.sync_copy(hp_hbm.at[pl.ds((NS - 1) * RPT, N - (NS - 1) * RPT)],
                        acc_sh.at[pl.ds((NS - 1) * RPT, N - (NS - 1) * RPT)])
    plsc.subcore_barrier()

    idx_bufs = [(srcA_v, dstA_v), (srcB_v, dstB_v)]
    pltpu.async_copy(src_hbm.at[wid, 0], srcA_v, semi)
    pltpu.async_copy(dst_hbm.at[wid, 0], dstA_v, semi)
    for g in range(GR):
        src_v, dst_v = idx_bufs[g % 2]
        pltpu.make_async_copy(src_hbm.at[wid, g], src_v, semi).wait()
        pltpu.make_async_copy(dst_hbm.at[wid, g], dst_v, semi).wait()
        if g + 1 < GR:
            nsrc_v, ndst_v = idx_bufs[(g + 1) % 2]
            pltpu.async_copy(src_hbm.at[wid, g + 1], nsrc_v, semi)
            pltpu.async_copy(dst_hbm.at[wid, g + 1], ndst_v, semi)
        pltpu.async_copy(hp_hbm.at[src_v.at[0]], rows0_v, sem0)

        @pl.loop(0, CHG, step=2)
        def _(j):
            @pl.when(j + 1 < CHG)
            def _g1():
                pltpu.async_copy(hp_hbm.at[src_v.at[j + 1]], rows1_v, sem1)

            pltpu.make_async_copy(hp_hbm.at[src_v.at[j]], rows0_v, sem0).wait()
            pltpu.sync_copy(rows0_v, acc_sh.at[dst_v.at[j]], add=True)

            @pl.when(j + 2 < CHG)
            def _g0():
                pltpu.async_copy(hp_hbm.at[src_v.at[j + 2]], rows0_v, sem0)

            @pl.when(j + 1 < CHG)
            def _s1():
                pltpu.make_async_copy(hp_hbm.at[src_v.at[j + 1]], rows1_v, sem1).wait()
                pltpu.sync_copy(rows1_v, acc_sh.at[dst_v.at[j + 1]], add=True)

    plsc.subcore_barrier()
    pltpu.sync_copy(acc_sh.at[pl.ds(s * RPT, RPT)], out_hbm.at[c, pl.ds(s * RPT, RPT)])


_rows_call = functools.partial(
    pl.kernel,
    out_type=jax.ShapeDtypeStruct((NC, NP, D), jnp.float32),
    mesh=_mesh,
    scratch_types=[
        pltpu.VMEM((CHG, K), jnp.int32),
        pltpu.VMEM((CHG, K), jnp.int32),
        pltpu.VMEM((CHG, K), jnp.int32),
        pltpu.VMEM((CHG, K), jnp.int32),
        pltpu.VMEM((K, D), jnp.float32),
        pltpu.VMEM((K, D), jnp.float32),
        pltpu.VMEM_SHARED((NP, D), jnp.float32),
        pltpu.SemaphoreType.DMA,
        pltpu.SemaphoreType.DMA,
        pltpu.SemaphoreType.DMA,
    ],
)(_rows_body)


BLK = 2000
GRID = N // BLK


def _tc1_body(x_ref, w1_ref, degp_ref, hp_ref, dinv_ref):
    deg = degp_ref[0, :, 0:1] + degp_ref[1, :, 0:1] + 1.0
    dinv = lax.rsqrt(deg)
    z = jnp.dot(x_ref[...], w1_ref[...], preferred_element_type=jnp.float32)
    hp_ref[...] = z * dinv
    dinv_ref[...] = dinv


def _tc1(x, W1, degp):
    return pl.pallas_call(
        _tc1_body,
        grid=(GRID,),
        in_specs=[
            pl.BlockSpec((BLK, D), lambda i: (i, 0)),
            pl.BlockSpec((D, D), lambda i: (0, 0)),
            pl.BlockSpec((NC, BLK, DW), lambda i: (0, i, 0)),
        ],
        out_specs=[
            pl.BlockSpec((BLK, D), lambda i: (i, 0)),
            pl.BlockSpec((BLK, 1), lambda i: (i, 0)),
        ],
        out_shape=[
            jax.ShapeDtypeStruct((N, D), jnp.float32),
            jax.ShapeDtypeStruct((N, 1), jnp.float32),
        ],
    )(x, W1, degp)


def _tc2_body(s_ref, hp1_ref, dinv_ref, b1_ref, w2_ref, hp2_ref):
    dinv = dinv_ref[...]
    agg = (s_ref[0] + s_ref[1] - hp1_ref[...]) * dinv + b1_ref[...]
    h1 = jnp.maximum(agg, 0.0)
    z2 = jnp.dot(h1, w2_ref[...], preferred_element_type=jnp.float32)
    hp2_ref[...] = z2 * dinv


def _tc2(S1, hp1, dinv, b1, W2):
    return pl.pallas_call(
        _tc2_body,
        grid=(GRID,),
        in_specs=[
            pl.BlockSpec((NC, BLK, D), lambda i: (0, i, 0)),
            pl.BlockSpec((BLK, D), lambda i: (i, 0)),
            pl.BlockSpec((BLK, 1), lambda i: (i, 0)),
            pl.BlockSpec((1, D), lambda i: (0, 0)),
            pl.BlockSpec((D, D), lambda i: (0, 0)),
        ],
        out_specs=pl.BlockSpec((BLK, D), lambda i: (i, 0)),
        out_shape=jax.ShapeDtypeStruct((N, D), jnp.float32),
    )(S1, hp1, dinv, b1, W2)


def _tc3_body(s_ref, hp2_ref, dinv_ref, b2_ref, wmu_ref, bmu_ref, wlv_ref,
              blv_ref, mu_ref, lv_ref, gacc):
    i = pl.program_id(0)

    @pl.when(i == 0)
    def _init():
        gacc[...] = jnp.zeros_like(gacc)

    dinv = dinv_ref[...]
    agg = (s_ref[0] + s_ref[1] - hp2_ref[...]) * dinv + b2_ref[...]
    h2 = jnp.maximum(agg, 0.0)
    gacc[...] += jnp.sum(h2, axis=0, keepdims=True)

    @pl.when(i == pl.num_programs(0) - 1)
    def _fin():
        g = gacc[...] * (1.0 / N)
        mu_ref[...] = jnp.dot(g, wmu_ref[...], preferred_element_type=jnp.float32) + bmu_ref[...]
        lv_ref[...] = jnp.dot(g, wlv_ref[...], preferred_element_type=jnp.float32) + blv_ref[...]


def _tc3(S2, hp2, dinv, b2, Wmu, bmu, Wlv, blv):
    return pl.pallas_call(
        _tc3_body,
        grid=(GRID,),
        in_specs=[
            pl.BlockSpec((NC, BLK, D), lambda i: (0, i, 0)),
            pl.BlockSpec((BLK, D), lambda i: (i, 0)),
            pl.BlockSpec((BLK, 1), lambda i: (i, 0)),
            pl.BlockSpec((1, D), lambda i: (0, 0)),
            pl.BlockSpec((D, LAT), lambda i: (0, 0)),
            pl.BlockSpec((1, LAT), lambda i: (0, 0)),
            pl.BlockSpec((D, LAT), lambda i: (0, 0)),
            pl.BlockSpec((1, LAT), lambda i: (0, 0)),
        ],
        out_specs=[
            pl.BlockSpec((1, LAT), lambda i: (0, 0)),
            pl.BlockSpec((1, LAT), lambda i: (0, 0)),
        ],
        out_shape=[
            jax.ShapeDtypeStruct((1, LAT), jnp.float32),
            jax.ShapeDtypeStruct((1, LAT), jnp.float32),
        ],
        scratch_shapes=[pltpu.VMEM((1, D), jnp.float32)],
    )(S2, hp2, dinv, b2, Wmu, bmu, Wlv, blv)


def kernel(x, edge_index, W1, b1, W2, b2, Wmu, bmu, Wlv, blv):
    ei = edge_index.astype(jnp.int32)
    src3 = ei[0].reshape(NW, CH, K)
    dst3 = ei[1].reshape(NW, CH, K)
    src4 = ei[0].reshape(NW, GR, CHG, K)
    dst4 = ei[1].reshape(NW, GR, CHG, K)

    degp = _deg_call(dst3)
    hp1, dinv = _tc1(x, W1, degp)
    S1 = _rows_call(hp1, src4, dst4)
    hp2 = _tc2(S1, hp1, dinv, b1.reshape(1, D), W2)
    S2 = _rows_call(hp2, src4, dst4)
    mu, lv = _tc3(S2, hp2, dinv, b2.reshape(1, D), Wmu,
                  bmu.reshape(1, LAT), Wlv, blv.reshape(1, LAT))
    return mu.reshape(LAT), lv.reshape(LAT)
